# baseline (speedup 1.0000x reference)
import functools

import jax
import jax.numpy as jnp
from jax import lax
from jax.experimental import pallas as pl
from jax.experimental.pallas import tpu as pltpu
from jax.experimental.pallas import tpu_sc as plsc

NUM_CLASSES = 1000
BATCH = 16384
LANES = 16
NUM_WORKERS = 32
COLS_PER_W = BATCH // NUM_WORKERS
CHUNK_COLS = 128
NCHUNKS = COLS_PER_W // CHUNK_COLS

_mesh = plsc.VectorSubcoreMesh(core_axis_name="c", subcore_axis_name="s")


@functools.partial(
    pl.kernel,
    out_type=jax.ShapeDtypeStruct((NUM_CLASSES, BATCH), jnp.float32),
    mesh=_mesh,
    scratch_types=[
        pltpu.VMEM((NUM_CLASSES, CHUNK_COLS), jnp.float32),
        pltpu.VMEM((2, CHUNK_COLS), jnp.int32),
        pltpu.SemaphoreType.DMA,
        pltpu.SemaphoreType.DMA,
    ],
)
def _onehot_body(labels_hbm, out_hbm, tmpl, lab2, sem, lab_sem):
    cid = lax.axis_index("c")
    sid = lax.axis_index("s")
    col0 = (cid * (NUM_WORKERS // 2) + sid) * COLS_PER_W

    def load_labels(c):
        pltpu.async_copy(
            labels_hbm.at[pl.ds(col0 + c * CHUNK_COLS, CHUNK_COLS)],
            lab2.at[c % 2], lab_sem)

    def wait_labels(c):
        pltpu.make_async_copy(
            labels_hbm.at[pl.ds(col0 + c * CHUNK_COLS, CHUNK_COLS)],
            lab2.at[c % 2], lab_sem).wait()

    load_labels(0)

    minus_ten = jnp.full((LANES,), -10.0, jnp.float32)
    iota16 = lax.iota(jnp.int32, LANES)

    def fill_row(r, carry):
        for g in range(CHUNK_COLS // LANES):
            tmpl[r, pl.ds(g * LANES, LANES)] = minus_ten
        return carry
    lax.fori_loop(0, NUM_CLASSES, fill_row, 0, unroll=2)

    def place(c):
        def elem(k, carry):
            goff = pl.multiple_of((k // LANES) * LANES, LANES)
            e = k % LANES
            rt = lab2[c % 2, pl.ds(k, 1)][0]
            old = tmpl[rt, pl.ds(goff, LANES)]
            tmpl[rt, pl.ds(goff, LANES)] = jnp.where(
                iota16 == e, jnp.float32(10.0), old)
            return carry
        lax.fori_loop(0, CHUNK_COLS, elem, 0)

    def restore(c):
        def elem(k, carry):
            goff = pl.multiple_of((k // LANES) * LANES, LANES)
            rt = lab2[c % 2, pl.ds(k, 1)][0]
            tmpl[rt, pl.ds(goff, LANES)] = minus_ten
            return carry
        lax.fori_loop(0, CHUNK_COLS, elem, 0)

    def fire(c):
        pltpu.async_copy(
            tmpl,
            out_hbm.at[:, pl.ds(col0 + c * CHUNK_COLS, CHUNK_COLS)],
            sem)

    def drain():
        pltpu.make_async_copy(
            tmpl, out_hbm.at[:, pl.ds(0, CHUNK_COLS)], sem).wait()

    def chunk_body(c, carry):
        @pl.when(c > 0)
        def _drain_and_restore():
            drain()
            restore(c - 1)

        wait_labels(c)
        place(c)
        fire(c)

        @pl.when(c + 1 < NCHUNKS)
        def _prefetch():
            load_labels(c + 1)

        return carry

    lax.fori_loop(0, NCHUNKS, chunk_body, 0)

    drain()


def kernel(input_ids, dummy):
    labels = input_ids[:, 0].astype(jnp.int32)
    return _onehot_body(labels).T


# speedup vs baseline: 1.2310x; 1.2310x over previous
import functools

import jax
import jax.numpy as jnp
from jax import lax
from jax.experimental import pallas as pl
from jax.experimental.pallas import tpu as pltpu
from jax.experimental.pallas import tpu_sc as plsc

NUM_CLASSES = 1000
BATCH = 16384
LANES = 16
NUM_WORKERS = 32
COLS_PER_W = BATCH // NUM_WORKERS
CHUNK_COLS = 128
NCHUNKS = COLS_PER_W // CHUNK_COLS

_mesh = plsc.VectorSubcoreMesh(core_axis_name="c", subcore_axis_name="s")


@functools.partial(
    pl.kernel,
    out_type=jax.ShapeDtypeStruct((NUM_CLASSES, BATCH), jnp.float32),
    mesh=_mesh,
    scratch_types=[
        pltpu.VMEM((NUM_CLASSES, CHUNK_COLS), jnp.float32),
        pltpu.VMEM((2, CHUNK_COLS), jnp.int32),
        pltpu.SemaphoreType.DMA,
        pltpu.SemaphoreType.DMA,
        pltpu.SemaphoreType.DMA,
    ],
)
def _onehot_body(labels_hbm, out_hbm, tmpl, lab2, sem, sem2, lab_sem):
    cid = lax.axis_index("c")
    sid = lax.axis_index("s")
    col0 = (cid * (NUM_WORKERS // 2) + sid) * COLS_PER_W

    def load_labels(c):
        pltpu.async_copy(
            labels_hbm.at[pl.ds(col0 + c * CHUNK_COLS, CHUNK_COLS)],
            lab2.at[c % 2], lab_sem)

    def wait_labels(c):
        pltpu.make_async_copy(
            labels_hbm.at[pl.ds(col0 + c * CHUNK_COLS, CHUNK_COLS)],
            lab2.at[c % 2], lab_sem).wait()

    load_labels(0)

    minus_ten = jnp.full((LANES,), -10.0, jnp.float32)
    iota16 = lax.iota(jnp.int32, LANES)

    def fill_row(r, carry):
        for g in range(CHUNK_COLS // LANES):
            tmpl[r, pl.ds(g * LANES, LANES)] = minus_ten
        return carry
    lax.fori_loop(0, NUM_CLASSES, fill_row, 0, unroll=4)

    def place(c):
        def group(g, carry):
            goff = pl.multiple_of(g * LANES, LANES)
            lab16 = lab2[c % 2, pl.ds(goff, LANES)]
            for e in range(LANES):
                rt = lab16[e]
                old = tmpl[rt, pl.ds(goff, LANES)]
                tmpl[rt, pl.ds(goff, LANES)] = jnp.where(
                    iota16 == e, jnp.float32(10.0), old)
            return carry
        lax.fori_loop(0, CHUNK_COLS // LANES, group, 0)

    def restore(c):
        def group(g, carry):
            goff = pl.multiple_of(g * LANES, LANES)
            lab16 = lab2[c % 2, pl.ds(goff, LANES)]
            for e in range(LANES):
                tmpl[lab16[e], pl.ds(goff, LANES)] = minus_ten
            return carry
        lax.fori_loop(0, CHUNK_COLS // LANES, group, 0)

    TOP = 504
    BOT = NUM_CLASSES - TOP

    def fire(c):
        start = col0 + c * CHUNK_COLS
        pltpu.async_copy(
            tmpl.at[pl.ds(0, TOP)],
            out_hbm.at[pl.ds(0, TOP), pl.ds(start, CHUNK_COLS)],
            sem)
        pltpu.async_copy(
            tmpl.at[pl.ds(TOP, BOT)],
            out_hbm.at[pl.ds(TOP, BOT), pl.ds(start, CHUNK_COLS)],
            sem2)

    def drain():
        pltpu.make_async_copy(
            tmpl.at[pl.ds(0, TOP)],
            out_hbm.at[pl.ds(0, TOP), pl.ds(0, CHUNK_COLS)], sem).wait()
        pltpu.make_async_copy(
            tmpl.at[pl.ds(TOP, BOT)],
            out_hbm.at[pl.ds(TOP, BOT), pl.ds(0, CHUNK_COLS)], sem2).wait()

    def chunk_body(c, carry):
        @pl.when(c > 0)
        def _drain_and_restore():
            drain()
            restore(c - 1)

        wait_labels(c)
        place(c)
        fire(c)

        @pl.when(c + 1 < NCHUNKS)
        def _prefetch():
            load_labels(c + 1)

        return carry

    lax.fori_loop(0, NCHUNKS, chunk_body, 0)

    drain()


def kernel(input_ids, dummy):
    labels = input_ids[:, 0].astype(jnp.int32)
    return _onehot_body(labels).T
